# baseline (speedup 1.0000x reference)
import functools

import jax
import jax.numpy as jnp
from jax import lax
from jax.experimental import pallas as pl
from jax.experimental.pallas import tpu as pltpu
from jax.experimental.pallas import tpu_sc as plsc

NUM_ENTITY = 1000000
NUM_RELATION = 1000
DIM = 32
PACK = 128 // DIM
MAXS = 12.0
BATCH = 16384

NC, NS, L = 2, 16, 16
NW = NC * NS
BPW = BATCH // NW
CHUNK = 128
NCHUNK = BPW // CHUNK
CGROUPS = CHUNK // L

EBLK = 128
GBLK = 4
NGROUP = (NUM_ENTITY // EBLK) // GBLK
GPW = NGROUP // NW
ITERS = GPW + 1
ROWS2 = NUM_ENTITY // PACK


def _format_body(ent_t, tail16, ent2, ibuf, obuf, sem_in, sem_out):
    w = lax.axis_index("s") * NC + lax.axis_index("c")
    last = jnp.where(w == NW - 1, ITERS - 1, GPW - 1)

    def g_of(j):
        return w * GPW + jnp.minimum(j, last)

    def fire_in(j, p):
        g = g_of(j)
        return [pltpu.async_copy(
            ent_t.at[:, pl.ds((g * GBLK + q) * EBLK, EBLK)],
            ibuf.at[p, q], sem_in) for q in range(GBLK)]

    in_cps = [fire_in(0, 0)]
    out_cps = []
    iota16 = lax.iota(jnp.int32, L)
    for j in range(ITERS):
        p = j % 2
        if j + 1 < ITERS:
            in_cps.append(fire_in(j + 1, 1 - p))
        for cp in in_cps[j]:
            cp.wait()
        if j >= 2:
            out_cps[j - 2].wait()
        pv = jnp.full((L,), p, jnp.int32)

        def rr_body(rr, carry):
            q = rr // 32
            col = (rr % 32) * PACK
            qv = jnp.full((L,), q, jnp.int32)
            for k in range(8):
                dv = 16 * (k % 2) + iota16
                cv = jnp.full((L,), col + k // 2, jnp.int32)
                v = plsc.load_gather(ibuf, [pv, qv, dv, cv])
                obuf[p, rr, pl.ds(16 * k, L)] = v
            return carry

        lax.fori_loop(0, 128, rr_body, 0)
        out_cps.append(pltpu.async_copy(
            obuf.at[p], ent2.at[pl.ds(g_of(j) * 128, 128)], sem_out))
    out_cps[-2].wait()
    out_cps[-1].wait()

    @pl.when(w == 0)
    def _tail():
        pltpu.sync_copy(tail16, obuf.at[0, pl.ds(0, 16)])
        pltpu.sync_copy(obuf.at[0, pl.ds(0, 16)],
                        ent2.at[pl.ds(NGROUP * 128, 16)])


def _score_body(ent2, rel2, hq, hrem, tq, trem, rq, rrem, out,
                hq_v, tq_v, hrem_v, trem_v, rq_v, rrem_v,
                h_buf, t_buf, rel_v, out_v, sem_a, sem_b):
    wid = lax.axis_index("s") * NC + lax.axis_index("c")

    pltpu.sync_copy(hq.at[wid], hq_v)
    pltpu.sync_copy(tq.at[wid], tq_v)
    pltpu.sync_copy(hrem.at[wid], hrem_v)
    pltpu.sync_copy(trem.at[wid], trem_v)
    pltpu.sync_copy(rq.at[wid], rq_v)
    pltpu.sync_copy(rrem.at[wid], rrem_v)

    rel_cp = pltpu.async_copy(rel2, rel_v, sem_b)

    def fire(c):
        p = c % 2
        return (pltpu.async_copy(ent2.at[hq_v.at[c]], h_buf.at[p], sem_a),
                pltpu.async_copy(ent2.at[tq_v.at[c]], t_buf.at[p], sem_a))

    def compute(c):
        p = c % 2
        pv = jnp.full((L,), p, jnp.int32)
        base = c * CHUNK
        for g in range(CGROUPS):
            b0 = base + g * L
            rows = g * L + lax.iota(jnp.int32, L)
            hoff = hrem_v[pl.ds(b0, L)]
            toff = trem_v[pl.ds(b0, L)]
            rqv = rq_v[pl.ds(b0, L)]
            roff = rrem_v[pl.ds(b0, L)]
            acc = jnp.zeros((L,), jnp.float32)
            for d in range(DIM):
                hv = plsc.load_gather(h_buf, [pv, rows, hoff + d])
                tv = plsc.load_gather(t_buf, [pv, rows, toff + d])
                rv = plsc.load_gather(rel_v, [rqv, roff + d])
                acc = acc + jnp.abs(hv + rv - tv)
            out_v[pl.ds(b0, L)] = MAXS - acc

    inflight = fire(0)
    rel_cp.wait()
    for c in range(NCHUNK):
        nxt = fire(c + 1) if c + 1 < NCHUNK else ()
        for cp in inflight:
            cp.wait()
        compute(c)
        inflight = nxt

    pltpu.sync_copy(out_v, out.at[pl.ds(wid * BPW, BPW)])


@jax.jit
def _transe_sc(ent_t, tail16, rel2, hq, hrem, tq, trem, rq, rrem):
    mesh = plsc.VectorSubcoreMesh(core_axis_name="c", subcore_axis_name="s",
                                  num_cores=NC, num_subcores=NS)
    ent2 = pl.kernel(
        _format_body,
        out_type=jax.ShapeDtypeStruct((ROWS2, 128), jnp.float32),
        mesh=mesh,
        scratch_types=[
            pltpu.VMEM((2, GBLK, DIM, EBLK), jnp.float32),
            pltpu.VMEM((2, 128, 128), jnp.float32),
            pltpu.SemaphoreType.DMA,
            pltpu.SemaphoreType.DMA,
        ],
        compiler_params=pltpu.CompilerParams(needs_layout_passes=False),
    )(ent_t, tail16)
    return pl.kernel(
        _score_body,
        out_type=jax.ShapeDtypeStruct((BATCH,), jnp.float32),
        mesh=mesh,
        scratch_types=[
            pltpu.VMEM((NCHUNK, CHUNK), jnp.int32),
            pltpu.VMEM((NCHUNK, CHUNK), jnp.int32),
            pltpu.VMEM((BPW,), jnp.int32),
            pltpu.VMEM((BPW,), jnp.int32),
            pltpu.VMEM((BPW,), jnp.int32),
            pltpu.VMEM((BPW,), jnp.int32),
            pltpu.VMEM((2, CHUNK, 128), jnp.float32),
            pltpu.VMEM((2, CHUNK, 128), jnp.float32),
            pltpu.VMEM((NUM_RELATION // PACK, 128), jnp.float32),
            pltpu.VMEM((BPW,), jnp.float32),
            pltpu.SemaphoreType.DMA,
            pltpu.SemaphoreType.DMA,
        ],
        compiler_params=pltpu.CompilerParams(needs_layout_passes=False),
    )(ent2, rel2, hq, hrem, tq, trem, rq, rrem)


def kernel(entity, relation, h_index, t_index, r_index, graph):
    h = h_index.astype(jnp.int32)
    t = t_index.astype(jnp.int32)
    r = r_index.astype(jnp.int32)
    hq = (h // PACK).reshape(NW, NCHUNK, CHUNK)
    tq = (t // PACK).reshape(NW, NCHUNK, CHUNK)
    hrem = ((h % PACK) * DIM).reshape(NW, BPW)
    trem = ((t % PACK) * DIM).reshape(NW, BPW)
    rq = (r // PACK).reshape(NW, BPW)
    rrem = ((r % PACK) * DIM).reshape(NW, BPW)
    tail16 = entity[NGROUP * GBLK * EBLK:].reshape(16, 128)
    rel2 = relation.reshape(NUM_RELATION // PACK, 128)
    return _transe_sc(entity.T, tail16, rel2, hq, hrem, tq, trem, rq, rrem)


# speedup vs baseline: 1.6017x; 1.6017x over previous
import functools

import jax
import jax.numpy as jnp
from jax import lax
from jax.experimental import pallas as pl
from jax.experimental.pallas import tpu as pltpu
from jax.experimental.pallas import tpu_sc as plsc

NUM_ENTITY = 1000000
NUM_RELATION = 1000
DIM = 32
MAXS = 12.0
BATCH = 16384

NC, NS, L = 2, 16, 16
NW = NC * NS
BPW = BATCH // NW
CHUNK = 128
NCHUNK = BPW // CHUNK


def _body(entity, relation, hidx, tidx, ridx, out,
          hidx_v, tidx_v, ridx_v, hr_buf, t_buf, out_v, sem_a, sem_b):
    wid = lax.axis_index("s") * NC + lax.axis_index("c")
    base = wid * BPW

    pltpu.sync_copy(hidx.at[wid], hidx_v)
    pltpu.sync_copy(tidx.at[wid], tidx_v)
    pltpu.sync_copy(ridx.at[wid], ridx_v)

    hr2d = hr_buf
    t2d = t_buf
    h_cps = []
    t_cps = []
    for c in range(NCHUNK):
        h_cps.append(pltpu.async_copy(
            entity.at[hidx_v.at[c]], hr2d.at[pl.ds(c * CHUNK, CHUNK)], sem_a))
        t_cps.append(pltpu.async_copy(
            entity.at[tidx_v.at[c]], t2d.at[pl.ds(c * CHUNK, CHUNK)], sem_b))
    for cp in h_cps:
        cp.wait()
    r_cps = []
    for c in range(NCHUNK):
        r_cps.append(pltpu.async_copy(
            relation.at[ridx_v.at[c]], hr2d.at[pl.ds(c * CHUNK, CHUNK)],
            sem_a, add=True))
    for cp in r_cps:
        cp.wait()
    for cp in t_cps:
        cp.wait()

    def group_body(g, carry):
        rows = g * L + lax.iota(jnp.int32, L)
        acc = jnp.zeros((L,), jnp.float32)
        for d in range(DIM):
            col = jnp.full((L,), d, jnp.int32)
            hr = plsc.load_gather(hr_buf, [rows, col])
            t = plsc.load_gather(t_buf, [rows, col])
            acc = acc + jnp.abs(hr - t)
        out_v[pl.ds(g * L, L)] = MAXS - acc
        return carry

    lax.fori_loop(0, BPW // L, group_body, 0)

    pltpu.sync_copy(out_v, out.at[pl.ds(base, BPW)])


@jax.jit
def _transe_sc(entity, relation, hidx, tidx, ridx):
    mesh = plsc.VectorSubcoreMesh(core_axis_name="c", subcore_axis_name="s",
                                  num_cores=NC, num_subcores=NS)
    return pl.kernel(
        _body,
        out_type=jax.ShapeDtypeStruct((BATCH,), jnp.float32),
        mesh=mesh,
        scratch_types=[
            pltpu.VMEM((NCHUNK, CHUNK), jnp.int32),
            pltpu.VMEM((NCHUNK, CHUNK), jnp.int32),
            pltpu.VMEM((NCHUNK, CHUNK), jnp.int32),
            pltpu.VMEM((BPW, DIM), jnp.float32),
            pltpu.VMEM((BPW, DIM), jnp.float32),
            pltpu.VMEM((BPW,), jnp.float32),
            pltpu.SemaphoreType.DMA,
            pltpu.SemaphoreType.DMA,
        ],
        compiler_params=pltpu.CompilerParams(needs_layout_passes=False,
                                             use_tc_tiling_on_sc=False),
    )(entity, relation, hidx, tidx, ridx)


def kernel(entity, relation, h_index, t_index, r_index, graph):
    h = h_index.astype(jnp.int32).reshape(NW, NCHUNK, CHUNK)
    t = t_index.astype(jnp.int32).reshape(NW, NCHUNK, CHUNK)
    r = r_index.astype(jnp.int32).reshape(NW, NCHUNK, CHUNK)
    return _transe_sc(entity, relation, h, t, r)
